# baseline (speedup 1.0000x reference)
import functools

import jax
import jax.numpy as jnp
from jax import lax
from jax.experimental import pallas as pl
from jax.experimental.pallas import tpu as pltpu
from jax.experimental.pallas import tpu_sc as plsc

B = 16384
NEG = 16
D = 64
L = 16
NC = 2
NS = 16
NW = NC * NS
PER_W = B // NW
CHUNK = 16
DP = 128
NCHUNK = PER_W // CHUNK
NROWS = CHUNK * NEG
NBUF = 2
PITCH = 17


def _body(ent_hbm, rel_hbm, h_hbm, r_hbm, t_hbm, tn_hbm, pos_hbm, neg_hbm,
          h_idx, r_idx, t_idx, n_idx, h_rows, r_rows, t_rows, n_rows,
          pos_all, neg_all, pscr, tscr, sems):
    wid = lax.axis_index("s") * NC + lax.axis_index("c")
    base = pl.multiple_of(wid * PER_W, PER_W)
    nbase = pl.multiple_of(base * NEG, PER_W * NEG)

    pltpu.sync_copy(h_hbm.at[pl.ds(base, PER_W)], h_idx)
    pltpu.sync_copy(r_hbm.at[pl.ds(base, PER_W)], r_idx)
    pltpu.sync_copy(t_hbm.at[pl.ds(base, PER_W)], t_idx)
    pltpu.sync_copy(tn_hbm.at[pl.ds(nbase, PER_W * NEG)], n_idx)

    iota = lax.iota(jnp.int32, L)
    iota_p = iota * PITCH
    slices = [pl.ds(dl * L, L) for dl in range(D // L)]

    def fire(c, b):
        sl = pl.ds(pl.multiple_of(c * CHUNK, CHUNK), CHUNK)
        nsl = pl.ds(pl.multiple_of(c * NROWS, NROWS), NROWS)
        pltpu.async_copy(ent_hbm.at[h_idx.at[sl]], h_rows.at[b], sems.at[b])
        pltpu.async_copy(rel_hbm.at[r_idx.at[sl]], r_rows.at[b], sems.at[b])
        pltpu.async_copy(ent_hbm.at[t_idx.at[sl]], t_rows.at[b], sems.at[b])
        pltpu.async_copy(ent_hbm.at[n_idx.at[nsl]], n_rows.at[b], sems.at[b])

    def wait_buf(b):
        sl = pl.ds(0, CHUNK)
        nsl = pl.ds(0, NROWS)
        pltpu.make_async_copy(ent_hbm.at[h_idx.at[sl]], h_rows.at[b], sems.at[b]).wait()
        pltpu.make_async_copy(rel_hbm.at[r_idx.at[sl]], r_rows.at[b], sems.at[b]).wait()
        pltpu.make_async_copy(ent_hbm.at[t_idx.at[sl]], t_rows.at[b], sems.at[b]).wait()
        pltpu.make_async_copy(ent_hbm.at[n_idx.at[nsl]], n_rows.at[b], sems.at[b]).wait()

    def l1_acc(hr, ref, row):
        acc = None
        for dl in range(D // L):
            term = jnp.abs(hr[dl] - ref[row, slices[dl]])
            acc = term if acc is None else acc + term
        return acc

    def col_sum(scr):
        acc = None
        for l in range(L):
            col = plsc.load_gather(scr, [iota_p + l])
            acc = col if acc is None else acc + col
        return acc

    def compute(c, b):
        hb, rb, tb, nb = h_rows.at[b], r_rows.at[b], t_rows.at[b], n_rows.at[b]
        for g in range(CHUNK // L):

            def item_body(j, carry):
                li = g * L + j
                hr = [hb[li, sl] + rb[li, sl] for sl in slices]
                plsc.store_scatter(pscr, [iota + j * PITCH], l1_acc(hr, tb, li))
                for n in range(NEG):
                    plsc.store_scatter(
                        tscr, [iota + n * PITCH], l1_acc(hr, nb, li * NEG + n))
                neg_all[c * CHUNK + li, :] = col_sum(tscr)
                return carry

            lax.fori_loop(0, L, item_body, 0)
            pos_all[pl.ds(pl.multiple_of(c * CHUNK + g * L, L), L)] = col_sum(pscr)

    fire(0, 0)

    def pair_body(k, carry):
        c0 = k * 2
        fire(c0 + 1, 1)
        wait_buf(0)
        compute(c0, 0)

        @pl.when(k < NCHUNK // 2 - 1)
        def _():
            fire(c0 + 2, 0)

        wait_buf(1)
        compute(c0 + 1, 1)
        return carry

    lax.fori_loop(0, NCHUNK // 2, pair_body, 0)

    pltpu.sync_copy(pos_all, pos_hbm.at[pl.ds(base, PER_W)])
    pltpu.sync_copy(neg_all, neg_hbm.at[pl.ds(base, PER_W)])


@jax.jit
def _transe(h_ids, r_ids, t_ids, tn_flat, entity_emb, relation_emb):
    mesh = plsc.VectorSubcoreMesh(core_axis_name="c", subcore_axis_name="s")
    run = functools.partial(
        pl.kernel,
        mesh=mesh,
        compiler_params=pltpu.CompilerParams(
            needs_layout_passes=False, use_tc_tiling_on_sc=False),
        out_type=[
            jax.ShapeDtypeStruct((B,), jnp.float32),
            jax.ShapeDtypeStruct((B, NEG), jnp.float32),
        ],
        scratch_types=[
            pltpu.VMEM((PER_W,), jnp.int32),
            pltpu.VMEM((PER_W,), jnp.int32),
            pltpu.VMEM((PER_W,), jnp.int32),
            pltpu.VMEM((PER_W * NEG,), jnp.int32),
            pltpu.VMEM((NBUF, CHUNK, DP), jnp.float32),
            pltpu.VMEM((NBUF, CHUNK, DP), jnp.float32),
            pltpu.VMEM((NBUF, CHUNK, DP), jnp.float32),
            pltpu.VMEM((NBUF, NROWS, DP), jnp.float32),
            pltpu.VMEM((PER_W,), jnp.float32),
            pltpu.VMEM((PER_W, NEG), jnp.float32),
            pltpu.VMEM((L * PITCH,), jnp.float32),
            pltpu.VMEM((L * PITCH,), jnp.float32),
            pltpu.SemaphoreType.DMA((NBUF,)),
        ],
    )(_body)
    return run(entity_emb, relation_emb, h_ids, r_ids, t_ids, tn_flat)


def kernel(h_ids, r_ids, t_ids, t_neg_ids, entity_emb, relation_emb):
    tn_flat = t_neg_ids.astype(jnp.int32).reshape(B * NEG)
    ent_pad = entity_emb.reshape(500000, DP)
    rel_pad = relation_emb.reshape(500, DP)
    pos, neg = _transe(jnp.right_shift(h_ids.astype(jnp.int32), 1),
                       jnp.right_shift(r_ids.astype(jnp.int32), 1),
                       jnp.right_shift(t_ids.astype(jnp.int32), 1),
                       jnp.right_shift(tn_flat, 1),
                       ent_pad, rel_pad)
    return pos, neg


# speedup vs baseline: 1.0987x; 1.0987x over previous
import functools

import jax
import jax.numpy as jnp
from jax import lax
from jax.experimental import pallas as pl
from jax.experimental.pallas import tpu as pltpu
from jax.experimental.pallas import tpu_sc as plsc

B = 16384
NEG = 16
D = 64
L = 16
NC = 2
NS = 16
NW = NC * NS
PER_W = B // NW
CHUNK = 16
DP = 128
NCHUNK = PER_W // CHUNK
NROWS = CHUNK * NEG
NBUF = 2
PITCH = 17


def _body(ent_hbm, rel_hbm, h_hbm, r_hbm, t_hbm, tn_hbm, pos_hbm, neg_hbm,
          h_idx, r_idx, t_idx, n_idx, h_rows, r_rows, t_rows, n_rows,
          pos_all, neg_all, pscr, tscr, sems):
    wid = lax.axis_index("s") * NC + lax.axis_index("c")
    base = pl.multiple_of(wid * PER_W, PER_W)
    nbase = pl.multiple_of(base * NEG, PER_W * NEG)

    pltpu.sync_copy(h_hbm.at[pl.ds(base, PER_W)], h_idx)
    pltpu.sync_copy(r_hbm.at[pl.ds(base, PER_W)], r_idx)
    pltpu.sync_copy(t_hbm.at[pl.ds(base, PER_W)], t_idx)
    pltpu.sync_copy(tn_hbm.at[pl.ds(nbase, PER_W * NEG)], n_idx)

    iota = lax.iota(jnp.int32, L)
    iota_p = iota * PITCH
    slices = [pl.ds(dl * L, L) for dl in range(D // L)]

    def fire(c, b):
        sl = pl.ds(pl.multiple_of(c * CHUNK, CHUNK), CHUNK)
        nsl = pl.ds(pl.multiple_of(c * NROWS, NROWS), NROWS)
        pltpu.async_copy(ent_hbm.at[h_idx.at[sl]], h_rows.at[b], sems.at[b])
        pltpu.async_copy(rel_hbm.at[r_idx.at[sl]], r_rows.at[b], sems.at[b])
        pltpu.async_copy(ent_hbm.at[t_idx.at[sl]], t_rows.at[b], sems.at[b])
        pltpu.async_copy(ent_hbm.at[n_idx.at[nsl]], n_rows.at[b], sems.at[b])

    def wait_buf(b):
        sl = pl.ds(0, CHUNK)
        nsl = pl.ds(0, NROWS)
        pltpu.make_async_copy(ent_hbm.at[h_idx.at[sl]], h_rows.at[b], sems.at[b]).wait()
        pltpu.make_async_copy(rel_hbm.at[r_idx.at[sl]], r_rows.at[b], sems.at[b]).wait()
        pltpu.make_async_copy(ent_hbm.at[t_idx.at[sl]], t_rows.at[b], sems.at[b]).wait()
        pltpu.make_async_copy(ent_hbm.at[n_idx.at[nsl]], n_rows.at[b], sems.at[b]).wait()

    def l1_acc(hr, ref, row):
        acc = None
        for dl in range(D // L):
            term = jnp.abs(hr[dl] - ref[row, slices[dl]])
            acc = term if acc is None else acc + term
        return acc

    def col_sum(scr):
        acc = None
        for l in range(L):
            col = plsc.load_gather(scr, [iota_p + l])
            acc = col if acc is None else acc + col
        return acc

    def compute(c, b):
        hb, rb, tb, nb = h_rows.at[b], r_rows.at[b], t_rows.at[b], n_rows.at[b]
        for g in range(CHUNK // L):

            def item_body(j, carry):
                li = g * L + j
                hr = [hb[li, sl] + rb[li, sl] for sl in slices]
                plsc.store_scatter(pscr, [iota + j * PITCH], l1_acc(hr, tb, li))
                for n in range(NEG):
                    plsc.store_scatter(
                        tscr, [iota + n * PITCH], l1_acc(hr, nb, li * NEG + n))
                neg_all[c * CHUNK + li, :] = col_sum(tscr)
                return carry

            lax.fori_loop(0, L, item_body, 0)
            pos_all[pl.ds(pl.multiple_of(c * CHUNK + g * L, L), L)] = col_sum(pscr)

    fire(0, 0)

    def pair_body(k, carry):
        c0 = k * 2
        fire(c0 + 1, 1)
        wait_buf(0)
        compute(c0, 0)

        @pl.when(k < NCHUNK // 2 - 1)
        def _():
            fire(c0 + 2, 0)

        wait_buf(1)
        compute(c0 + 1, 1)
        return carry

    lax.fori_loop(0, NCHUNK // 2, pair_body, 0)

    pltpu.sync_copy(pos_all, pos_hbm.at[pl.ds(base, PER_W)])
    pltpu.sync_copy(neg_all, neg_hbm.at[pl.ds(base, PER_W)])


@jax.jit
def _transe(h_ids, r_ids, t_ids, tn_flat, entity_emb, relation_emb):
    mesh = plsc.VectorSubcoreMesh(core_axis_name="c", subcore_axis_name="s")
    run = functools.partial(
        pl.kernel,
        mesh=mesh,
        compiler_params=pltpu.CompilerParams(
            needs_layout_passes=False, use_tc_tiling_on_sc=False),
        out_type=[
            jax.ShapeDtypeStruct((B,), jnp.float32),
            jax.ShapeDtypeStruct((B, NEG), jnp.float32),
        ],
        scratch_types=[
            pltpu.VMEM((PER_W,), jnp.int32),
            pltpu.VMEM((PER_W,), jnp.int32),
            pltpu.VMEM((PER_W,), jnp.int32),
            pltpu.VMEM((PER_W * NEG,), jnp.int32),
            pltpu.VMEM((NBUF, CHUNK, DP), jnp.float32),
            pltpu.VMEM((NBUF, CHUNK, DP), jnp.float32),
            pltpu.VMEM((NBUF, CHUNK, DP), jnp.float32),
            pltpu.VMEM((NBUF, NROWS, DP), jnp.float32),
            pltpu.VMEM((PER_W,), jnp.float32),
            pltpu.VMEM((PER_W, NEG), jnp.float32),
            pltpu.VMEM((L * PITCH,), jnp.float32),
            pltpu.VMEM((L * PITCH,), jnp.float32),
            pltpu.SemaphoreType.DMA((NBUF,)),
        ],
    )(_body)
    return run(entity_emb, relation_emb, h_ids, r_ids, t_ids, tn_flat)


def kernel(h_ids, r_ids, t_ids, t_neg_ids, entity_emb, relation_emb):
    tn_flat = t_neg_ids.astype(jnp.int32).reshape(B * NEG)
    ent_pad = jnp.pad(entity_emb, ((0, 0), (0, DP - D)))
    rel_pad = jnp.pad(relation_emb, ((0, 0), (0, DP - D)))
    pos, neg = _transe(h_ids.astype(jnp.int32), r_ids.astype(jnp.int32),
                       t_ids.astype(jnp.int32), tn_flat,
                       ent_pad, rel_pad)
    return pos, neg
